# baseline (speedup 1.0000x reference)
import jax
import jax.numpy as jnp
from jax import lax
from jax.experimental import pallas as pl
from jax.experimental.pallas import tpu as pltpu
from jax.experimental.pallas import tpu_sc as plsc

N, E, DIN, DH, DE, NCLS, NL, NG = 10000, 320000, 128, 256, 16, 10, 3, 16

DHH = DH // 2
NSUB = 16
EPT = E // NSUB
CH = 40
NPAIR = EPT // (2 * CH)
NPAD = 10240
RPT = NPAD // NSUB


def _ln(x, g, b, eps=1e-5):
    m = x.mean(-1, keepdims=True)
    v = ((x - m) ** 2).mean(-1, keepdims=True)
    return (x - m) / jnp.sqrt(v + eps) * g + b


def _bn(x, g, b, eps=1e-5):
    m = x.mean(0)
    v = ((x - m) ** 2).mean(0)
    return (x - m) / jnp.sqrt(v + eps) * g + b


def _edge_body(srch, dsth, ab, ex, acc, bufs, sub):
    sidx, didx, abb, exb, msg, semab, semex = bufs
    tbase = sub * EPT

    def issue(base, k):
        pltpu.sync_copy(srch.at[pl.ds(base, CH)], sidx[k])
        pltpu.sync_copy(dsth.at[pl.ds(base, CH)], didx[k])
        pltpu.async_copy(ab.at[sidx[k]], abb[k], semab[k])
        pltpu.async_copy(ex.at[pl.ds(base, CH)], exb[k], semex[k])

    def drain(k):
        pltpu.make_async_copy(ab.at[sidx[k]], abb[k], semab[k]).wait()
        pltpu.make_async_copy(ex.at[pl.ds(0, CH)], exb[k], semex[k]).wait()

    def consume(k):
        @plsc.parallel_loop(0, CH, 1, unroll=4)
        def row(i):
            for j in range(DHH // 16):
                a = abb[k][i, pl.ds(j * 16, 16)]
                b = abb[k][i, pl.ds(DHH + j * 16, 16)]
                cde = exb[k][i, pl.ds(j * 16, 16)]
                s = 1.0 / (1.0 + jnp.exp(-(b + cde)))
                msg[k][i, pl.ds(j * 16, 16)] = a * s
        pltpu.sync_copy(msg[k], acc.at[didx[k]], add=True)

    issue(tbase, 0)

    def pair(p, _):
        base = tbase + 2 * p * CH
        issue(base + CH, 1)
        drain(0)
        consume(0)

        @pl.when(p < NPAIR - 1)
        def _():
            issue(base + 2 * CH, 0)

        drain(1)
        consume(1)
        return 0

    lax.fori_loop(0, NPAIR, pair, 0)


def _edge_sc(srch, dsth, ab0, ab1, ex0, ex1, agg0, agg1,
             acc, sidx0, didx0, abb0, exb0, msg0,
             sidx1, didx1, abb1, exb1, msg1,
             semab0, semex0, semab1, semex1):
    cid = lax.axis_index("c")
    sub = lax.axis_index("s")
    bufs = ((sidx0, sidx1), (didx0, didx1), (abb0, abb1), (exb0, exb1),
            (msg0, msg1), (semab0, semab1), (semex0, semex1))

    def zrow(i, _):
        for j in range(DHH // 16):
            msg0[i, pl.ds(j * 16, 16)] = jnp.zeros((16,), jnp.float32)
        return 0
    lax.fori_loop(0, CH, zrow, 0)
    for j in range(RPT // CH):
        pltpu.sync_copy(msg0, acc.at[pl.ds(sub * RPT + j * CH, CH)])
    plsc.subcore_barrier()

    @pl.when(cid == 0)
    def _():
        _edge_body(srch, dsth, ab0, ex0, acc, bufs, sub)

    @pl.when(cid == 1)
    def _():
        _edge_body(srch, dsth, ab1, ex1, acc, bufs, sub)

    plsc.subcore_barrier()

    @pl.when(cid == 0)
    def _():
        pltpu.sync_copy(acc.at[pl.ds(sub * RPT, RPT)],
                        agg0.at[pl.ds(sub * RPT, RPT)])

    @pl.when(cid == 1)
    def _():
        pltpu.sync_copy(acc.at[pl.ds(sub * RPT, RPT)],
                        agg1.at[pl.ds(sub * RPT, RPT)])


_edge_call = pl.kernel(
    _edge_sc,
    out_type=(jax.ShapeDtypeStruct((NPAD, DHH), jnp.float32),
              jax.ShapeDtypeStruct((NPAD, DHH), jnp.float32)),
    mesh=plsc.VectorSubcoreMesh(core_axis_name="c", subcore_axis_name="s"),
    compiler_params=pltpu.CompilerParams(use_tc_tiling_on_sc=False),
    scratch_types=[
        pltpu.VMEM_SHARED((NPAD, DHH), jnp.float32),
        pltpu.VMEM((CH,), jnp.int32),
        pltpu.VMEM((CH,), jnp.int32),
        pltpu.VMEM((CH, 2 * DHH), jnp.float32),
        pltpu.VMEM((CH, DHH), jnp.float32),
        pltpu.VMEM((CH, DHH), jnp.float32),
        pltpu.VMEM((CH,), jnp.int32),
        pltpu.VMEM((CH,), jnp.int32),
        pltpu.VMEM((CH, 2 * DHH), jnp.float32),
        pltpu.VMEM((CH, DHH), jnp.float32),
        pltpu.VMEM((CH, DHH), jnp.float32),
        pltpu.SemaphoreType.DMA,
        pltpu.SemaphoreType.DMA,
        pltpu.SemaphoreType.DMA,
        pltpu.SemaphoreType.DMA,
    ],
)


def kernel(x, edge_index, edge_attr, batch, params):
    p = params
    src, dst = edge_index[0], edge_index[1]
    t = jax.nn.relu(edge_attr @ p['e2n_W'] + p['e2n_b'])
    t = _ln(t, p['e2n_g'], p['e2n_be'])
    nf = jnp.zeros((N, DIN), jnp.float32).at[dst].add(t).at[src].add(t)
    deg = jnp.zeros((N,), jnp.float32).at[src].add(1.0).at[dst].add(1.0)
    nf = nf / jnp.maximum(deg, 1.0)[:, None]
    h = (x + nf) @ p['emb_W'] + p['emb_b']
    for i in range(NL):
        Ax = h @ p['WA'][i] + p['bA'][i]
        Bx = h @ p['WB'][i] + p['bB'][i]
        Cx = h @ p['WC'][i] + p['bC'][i]
        Dx = h @ p['WD'][i] + p['bD'][i]
        Ex = edge_attr @ p['WE'][i] + p['bE'][i]
        ce0 = Ex[:, :DHH] + Cx[dst, :DHH]
        ce1 = Ex[:, DHH:] + Cx[dst, DHH:]
        ab0 = jnp.concatenate([Ax[:, :DHH], Bx[:, :DHH]], axis=1)
        ab1 = jnp.concatenate([Ax[:, DHH:], Bx[:, DHH:]], axis=1)
        agg0, agg1 = _edge_call(src, dst, ab0, ab1, ce0, ce1)
        agg = jnp.concatenate([agg0[:N], agg1[:N]], axis=1)
        h = jax.nn.relu(_bn(agg * jax.nn.sigmoid(Dx) + h, p['bn_g'][i], p['bn_b'][i]))
    d = jnp.abs(h[src] - h[dst])
    ep = jax.nn.relu(d @ p['dec_W1'] + p['dec_b1']) @ p['dec_W2'] + p['dec_b2']
    adj_pred = jax.nn.sigmoid(ep)[:, 0]
    gsum = jax.ops.segment_sum(h, batch, num_segments=NG)
    gcnt = jax.ops.segment_sum(jnp.ones((N,), jnp.float32), batch, num_segments=NG)
    gemb = gsum / jnp.maximum(gcnt, 1.0)[:, None]
    class_logits = jax.nn.relu(gemb @ p['cls_W1'] + p['cls_b1']) @ p['cls_W2'] + p['cls_b2']
    return (adj_pred, class_logits, h)


# speedup vs baseline: 35.8262x; 35.8262x over previous
import jax
import jax.numpy as jnp
from jax.experimental import pallas as pl

N, E, DIN, DH, DE, NCLS, NL, NG = 10000, 320000, 128, 256, 16, 10, 3, 16
NB = 2000
EB = 4000


def _edge_mlp_kernel(ea_ref, w_ref, b_ref, g_ref, be_ref, t_ref):
    t = jax.nn.relu(jnp.dot(ea_ref[...], w_ref[...],
                            preferred_element_type=jnp.float32) + b_ref[...])
    m = t.mean(-1, keepdims=True)
    v = ((t - m) ** 2).mean(-1, keepdims=True)
    t_ref[...] = (t - m) / jnp.sqrt(v + 1e-5) * g_ref[...] + be_ref[...]


def _edge_mlp(ea, p):
    return pl.pallas_call(
        _edge_mlp_kernel,
        grid=(E // EB,),
        in_specs=[pl.BlockSpec((EB, DE), lambda i: (i, 0)),
                  pl.BlockSpec((DE, DIN), lambda i: (0, 0)),
                  pl.BlockSpec((DIN,), lambda i: (0,)),
                  pl.BlockSpec((DIN,), lambda i: (0,)),
                  pl.BlockSpec((DIN,), lambda i: (0,))],
        out_specs=pl.BlockSpec((EB, DIN), lambda i: (i, 0)),
        out_shape=jax.ShapeDtypeStruct((E, DIN), jnp.float32),
    )(ea, p['e2n_W'], p['e2n_b'], p['e2n_g'], p['e2n_be'])


def _emb_kernel(x_ref, nf_ref, deg_ref, w_ref, b_ref, h_ref):
    nf = nf_ref[...] / jnp.maximum(deg_ref[...], 1.0)
    h_ref[...] = (jnp.dot(x_ref[...] + nf, w_ref[...],
                          preferred_element_type=jnp.float32) + b_ref[...])


def _emb(x, nf, degb, p):
    return pl.pallas_call(
        _emb_kernel,
        grid=(N // NB,),
        in_specs=[pl.BlockSpec((NB, DIN), lambda i: (i, 0)),
                  pl.BlockSpec((NB, DIN), lambda i: (i, 0)),
                  pl.BlockSpec((NB, DIN), lambda i: (i, 0)),
                  pl.BlockSpec((DIN, DH), lambda i: (0, 0)),
                  pl.BlockSpec((DH,), lambda i: (0,))],
        out_specs=pl.BlockSpec((NB, DH), lambda i: (i, 0)),
        out_shape=jax.ShapeDtypeStruct((N, DH), jnp.float32),
    )(x, nf, degb, p['emb_W'], p['emb_b'])


def _abcd_kernel(h_ref, wa, wb, wc, wd, ba, bb, bc, bd,
                 a_ref, b_ref, c_ref, d_ref):
    h = h_ref[...]
    a_ref[...] = jnp.dot(h, wa[...], preferred_element_type=jnp.float32) + ba[...]
    b_ref[...] = jnp.dot(h, wb[...], preferred_element_type=jnp.float32) + bb[...]
    c_ref[...] = jnp.dot(h, wc[...], preferred_element_type=jnp.float32) + bc[...]
    d_ref[...] = jnp.dot(h, wd[...], preferred_element_type=jnp.float32) + bd[...]


def _abcd(h, p, i):
    wspec = pl.BlockSpec((DH, DH), lambda i: (0, 0))
    bspec = pl.BlockSpec((DH,), lambda i: (0,))
    nspec = pl.BlockSpec((NB, DH), lambda i: (i, 0))
    return pl.pallas_call(
        _abcd_kernel,
        grid=(N // NB,),
        in_specs=[nspec, wspec, wspec, wspec, wspec,
                  bspec, bspec, bspec, bspec],
        out_specs=[nspec] * 4,
        out_shape=[jax.ShapeDtypeStruct((N, DH), jnp.float32)] * 4,
    )(h, p['WA'][i], p['WB'][i], p['WC'][i], p['WD'][i],
      p['bA'][i], p['bB'][i], p['bC'][i], p['bD'][i])


def _ce_kernel(ea_ref, cxd_ref, we_ref, be_ref, ce_ref):
    ce_ref[...] = (jnp.dot(ea_ref[...], we_ref[...],
                           preferred_element_type=jnp.float32)
                   + be_ref[...] + cxd_ref[...])


def _ce(ea, cxd, p, i):
    return pl.pallas_call(
        _ce_kernel,
        grid=(E // EB,),
        in_specs=[pl.BlockSpec((EB, DE), lambda i: (i, 0)),
                  pl.BlockSpec((EB, DH), lambda i: (i, 0)),
                  pl.BlockSpec((DE, DH), lambda i: (0, 0)),
                  pl.BlockSpec((DH,), lambda i: (0,))],
        out_specs=pl.BlockSpec((EB, DH), lambda i: (i, 0)),
        out_shape=jax.ShapeDtypeStruct((E, DH), jnp.float32),
    )(ea, cxd, p['WE'][i], p['bE'][i])


def _gate_kernel(as_ref, bs_ref, ce_ref, msg_ref):
    msg_ref[...] = as_ref[...] * jax.nn.sigmoid(bs_ref[...] + ce_ref[...])


def _gate(a_s, b_s, ce):
    spec = pl.BlockSpec((EB, DH), lambda i: (i, 0))
    return pl.pallas_call(
        _gate_kernel,
        grid=(E // EB,),
        in_specs=[spec, spec, spec],
        out_specs=spec,
        out_shape=jax.ShapeDtypeStruct((E, DH), jnp.float32),
    )(a_s, b_s, ce)


def _bn_kernel(agg_ref, dx_ref, h_ref, g_ref, b_ref, out_ref):
    u = agg_ref[...] * jax.nn.sigmoid(dx_ref[...]) + h_ref[...]
    m = u.mean(0)
    v = ((u - m) ** 2).mean(0)
    out_ref[...] = jax.nn.relu((u - m) / jnp.sqrt(v + 1e-5) * g_ref[...]
                               + b_ref[...])


def _bn_update(agg, dx, h, p, i):
    return pl.pallas_call(
        _bn_kernel,
        out_shape=jax.ShapeDtypeStruct((N, DH), jnp.float32),
    )(agg, dx, h, p['bn_g'][i], p['bn_b'][i])


def _dec_kernel(hs_ref, hd_ref, w1_ref, b1_ref, w2_ref, b2_ref, out_ref):
    d = jnp.abs(hs_ref[...] - hd_ref[...])
    hid = jax.nn.relu(jnp.dot(d, w1_ref[...],
                              preferred_element_type=jnp.float32) + b1_ref[...])
    ep = jnp.dot(hid, w2_ref[...], preferred_element_type=jnp.float32) + b2_ref[...]
    out_ref[...] = jax.nn.sigmoid(ep)


def _decoder(hs, hd, p):
    return pl.pallas_call(
        _dec_kernel,
        grid=(E // EB,),
        in_specs=[pl.BlockSpec((EB, DH), lambda i: (i, 0)),
                  pl.BlockSpec((EB, DH), lambda i: (i, 0)),
                  pl.BlockSpec((DH, DH), lambda i: (0, 0)),
                  pl.BlockSpec((DH,), lambda i: (0,)),
                  pl.BlockSpec((DH, 128), lambda i: (0, 0)),
                  pl.BlockSpec((128,), lambda i: (0,))],
        out_specs=pl.BlockSpec((EB, 128), lambda i: (i, 0)),
        out_shape=jax.ShapeDtypeStruct((E, 128), jnp.float32),
    )(hs, hd, p['dec_W1'], p['dec_b1'],
      jnp.pad(p['dec_W2'], ((0, 0), (0, 127))),
      jnp.pad(p['dec_b2'], (0, 127)))


def _pool_cls_kernel(h_ref, batch_ref, w1_ref, b1_ref, w2_ref, b2_ref, out_ref):
    batch = batch_ref[...]
    onehot = (batch == jax.lax.broadcasted_iota(jnp.int32, (1, NG), 1)
              ).astype(jnp.float32)
    gsum = jax.lax.dot_general(onehot, h_ref[...], (((0,), (0,)), ((), ())),
                               preferred_element_type=jnp.float32)
    gcnt = jnp.sum(onehot, axis=0)
    gemb = gsum / jnp.maximum(gcnt, 1.0)[:, None]
    hid = jax.nn.relu(
        jnp.dot(gemb, w1_ref[...], preferred_element_type=jnp.float32)
        + b1_ref[...])
    out_ref[...] = (jnp.dot(hid, w2_ref[...],
                            preferred_element_type=jnp.float32) + b2_ref[...])


def _pool_cls(h, batch, p):
    return pl.pallas_call(
        _pool_cls_kernel,
        out_shape=jax.ShapeDtypeStruct((NG, NCLS), jnp.float32),
    )(h, batch[:, None], p['cls_W1'], p['cls_b1'], p['cls_W2'], p['cls_b2'])


def kernel(x, edge_index, edge_attr, batch, params):
    p = params
    src, dst = edge_index[0], edge_index[1]
    t = _edge_mlp(edge_attr, p)
    nf = jnp.zeros((N, DIN), jnp.float32).at[dst].add(t).at[src].add(t)
    deg = jnp.zeros((N,), jnp.float32).at[src].add(1.0).at[dst].add(1.0)
    degb = jnp.broadcast_to(deg[:, None], (N, DIN))
    h = _emb(x, nf, degb, p)
    for i in range(NL):
        Ax, Bx, Cx, Dx = _abcd(h, p, i)
        ce = _ce(edge_attr, Cx[dst], p, i)
        msg = _gate(Ax[src], Bx[src], ce)
        agg = jnp.zeros_like(h).at[dst].add(msg)
        h = _bn_update(agg, Dx, h, p, i)
    adj_pred = _decoder(h[src], h[dst], p)[:, 0]
    class_logits = _pool_cls(h, batch, p)
    return (adj_pred, class_logits, h)


# speedup vs baseline: 37.2052x; 1.0385x over previous
import jax
import jax.numpy as jnp
from jax.experimental import pallas as pl

N, E, DIN, DH, DE, NCLS, NL, NG = 10000, 320000, 128, 256, 16, 10, 3, 16
NB = 2000
EB = 4000


def _edge_mlp_kernel(ea_ref, w_ref, b_ref, g_ref, be_ref, t_ref):
    t = jax.nn.relu(jnp.dot(ea_ref[...], w_ref[...],
                            preferred_element_type=jnp.float32) + b_ref[...])
    m = t.mean(-1, keepdims=True)
    v = ((t - m) ** 2).mean(-1, keepdims=True)
    t_ref[...] = (t - m) / jnp.sqrt(v + 1e-5) * g_ref[...] + be_ref[...]


def _edge_mlp(ea, p):
    return pl.pallas_call(
        _edge_mlp_kernel,
        grid=(E // EB,),
        in_specs=[pl.BlockSpec((EB, DE), lambda i: (i, 0)),
                  pl.BlockSpec((DE, DIN), lambda i: (0, 0)),
                  pl.BlockSpec((DIN,), lambda i: (0,)),
                  pl.BlockSpec((DIN,), lambda i: (0,)),
                  pl.BlockSpec((DIN,), lambda i: (0,))],
        out_specs=pl.BlockSpec((EB, DIN), lambda i: (i, 0)),
        out_shape=jax.ShapeDtypeStruct((E, DIN), jnp.float32),
    )(ea, p['e2n_W'], p['e2n_b'], p['e2n_g'], p['e2n_be'])


def _emb_kernel(x_ref, nf_ref, deg_ref, w_ref, b_ref, h_ref):
    nf = nf_ref[...] / jnp.maximum(deg_ref[...], 1.0)
    h_ref[...] = (jnp.dot(x_ref[...] + nf, w_ref[...],
                          preferred_element_type=jnp.float32) + b_ref[...])


def _emb(x, nf, degb, p):
    return pl.pallas_call(
        _emb_kernel,
        grid=(N // NB,),
        in_specs=[pl.BlockSpec((NB, DIN), lambda i: (i, 0)),
                  pl.BlockSpec((NB, DIN), lambda i: (i, 0)),
                  pl.BlockSpec((NB, DIN), lambda i: (i, 0)),
                  pl.BlockSpec((DIN, DH), lambda i: (0, 0)),
                  pl.BlockSpec((DH,), lambda i: (0,))],
        out_specs=pl.BlockSpec((NB, DH), lambda i: (i, 0)),
        out_shape=jax.ShapeDtypeStruct((N, DH), jnp.float32),
    )(x, nf, degb, p['emb_W'], p['emb_b'])


def _abcd_kernel(h_ref, wa, wb, wc, wd, ba, bb, bc, bd,
                 a_ref, b_ref, c_ref, d_ref):
    h = h_ref[...]
    a_ref[...] = jnp.dot(h, wa[...], preferred_element_type=jnp.float32) + ba[...]
    b_ref[...] = jnp.dot(h, wb[...], preferred_element_type=jnp.float32) + bb[...]
    c_ref[...] = jnp.dot(h, wc[...], preferred_element_type=jnp.float32) + bc[...]
    d_ref[...] = jnp.dot(h, wd[...], preferred_element_type=jnp.float32) + bd[...]


def _abcd(h, p, i):
    wspec = pl.BlockSpec((DH, DH), lambda i: (0, 0))
    bspec = pl.BlockSpec((DH,), lambda i: (0,))
    nspec = pl.BlockSpec((NB, DH), lambda i: (i, 0))
    return pl.pallas_call(
        _abcd_kernel,
        grid=(N // NB,),
        in_specs=[nspec, wspec, wspec, wspec, wspec,
                  bspec, bspec, bspec, bspec],
        out_specs=[nspec] * 4,
        out_shape=[jax.ShapeDtypeStruct((N, DH), jnp.float32)] * 4,
    )(h, p['WA'][i], p['WB'][i], p['WC'][i], p['WD'][i],
      p['bA'][i], p['bB'][i], p['bC'][i], p['bD'][i])


def _gate_kernel(ea_ref, cxd_ref, as_ref, bs_ref, we_ref, be_ref, msg_ref):
    ce = (jnp.dot(ea_ref[...], we_ref[...],
                  preferred_element_type=jnp.float32)
          + be_ref[...] + cxd_ref[...])
    msg_ref[...] = as_ref[...] * jax.nn.sigmoid(bs_ref[...] + ce)


def _gate(ea, cxd, a_s, b_s, p, i):
    spec = pl.BlockSpec((EB, DH), lambda i: (i, 0))
    return pl.pallas_call(
        _gate_kernel,
        grid=(E // EB,),
        in_specs=[pl.BlockSpec((EB, DE), lambda i: (i, 0)),
                  spec, spec, spec,
                  pl.BlockSpec((DE, DH), lambda i: (0, 0)),
                  pl.BlockSpec((DH,), lambda i: (0,))],
        out_specs=spec,
        out_shape=jax.ShapeDtypeStruct((E, DH), jnp.float32),
    )(ea, cxd, a_s, b_s, p['WE'][i], p['bE'][i])


def _bn_kernel(agg_ref, dx_ref, h_ref, g_ref, b_ref, out_ref):
    u = agg_ref[...] * jax.nn.sigmoid(dx_ref[...]) + h_ref[...]
    m = u.mean(0)
    v = ((u - m) ** 2).mean(0)
    out_ref[...] = jax.nn.relu((u - m) / jnp.sqrt(v + 1e-5) * g_ref[...]
                               + b_ref[...])


def _bn_update(agg, dx, h, p, i):
    return pl.pallas_call(
        _bn_kernel,
        out_shape=jax.ShapeDtypeStruct((N, DH), jnp.float32),
    )(agg, dx, h, p['bn_g'][i], p['bn_b'][i])


def _dec_kernel(hs_ref, hd_ref, w1_ref, b1_ref, w2_ref, b2_ref, out_ref):
    d = jnp.abs(hs_ref[...] - hd_ref[...])
    hid = jax.nn.relu(jnp.dot(d, w1_ref[...],
                              preferred_element_type=jnp.float32) + b1_ref[...])
    ep = jnp.dot(hid, w2_ref[...], preferred_element_type=jnp.float32) + b2_ref[...]
    out_ref[...] = jax.nn.sigmoid(ep)


def _decoder(hs, hd, p):
    return pl.pallas_call(
        _dec_kernel,
        grid=(E // EB,),
        in_specs=[pl.BlockSpec((EB, DH), lambda i: (i, 0)),
                  pl.BlockSpec((EB, DH), lambda i: (i, 0)),
                  pl.BlockSpec((DH, DH), lambda i: (0, 0)),
                  pl.BlockSpec((DH,), lambda i: (0,)),
                  pl.BlockSpec((DH, 128), lambda i: (0, 0)),
                  pl.BlockSpec((128,), lambda i: (0,))],
        out_specs=pl.BlockSpec((EB, 128), lambda i: (i, 0)),
        out_shape=jax.ShapeDtypeStruct((E, 128), jnp.float32),
    )(hs, hd, p['dec_W1'], p['dec_b1'],
      jnp.pad(p['dec_W2'], ((0, 0), (0, 127))),
      jnp.pad(p['dec_b2'], (0, 127)))


def _pool_cls_kernel(h_ref, batch_ref, w1_ref, b1_ref, w2_ref, b2_ref, out_ref):
    batch = batch_ref[...]
    onehot = (batch == jax.lax.broadcasted_iota(jnp.int32, (1, NG), 1)
              ).astype(jnp.float32)
    gsum = jax.lax.dot_general(onehot, h_ref[...], (((0,), (0,)), ((), ())),
                               preferred_element_type=jnp.float32)
    gcnt = jnp.sum(onehot, axis=0)
    gemb = gsum / jnp.maximum(gcnt, 1.0)[:, None]
    hid = jax.nn.relu(
        jnp.dot(gemb, w1_ref[...], preferred_element_type=jnp.float32)
        + b1_ref[...])
    out_ref[...] = (jnp.dot(hid, w2_ref[...],
                            preferred_element_type=jnp.float32) + b2_ref[...])


def _pool_cls(h, batch, p):
    return pl.pallas_call(
        _pool_cls_kernel,
        out_shape=jax.ShapeDtypeStruct((NG, NCLS), jnp.float32),
    )(h, batch[:, None], p['cls_W1'], p['cls_b1'], p['cls_W2'], p['cls_b2'])


def kernel(x, edge_index, edge_attr, batch, params):
    p = params
    src, dst = edge_index[0], edge_index[1]
    t = _edge_mlp(edge_attr, p)
    nf = jnp.zeros((N, DIN), jnp.float32).at[dst].add(t).at[src].add(t)
    deg = jnp.zeros((N,), jnp.float32).at[src].add(1.0).at[dst].add(1.0)
    degb = jnp.broadcast_to(deg[:, None], (N, DIN))
    h = _emb(x, nf, degb, p)
    for i in range(NL):
        Ax, Bx, Cx, Dx = _abcd(h, p, i)
        msg = _gate(edge_attr, Cx[dst], Ax[src], Bx[src], p, i)
        agg = jnp.zeros_like(h).at[dst].add(msg)
        h = _bn_update(agg, Dx, h, p, i)
    adj_pred = _decoder(h[src], h[dst], p)[:, 0]
    class_logits = _pool_cls(h, batch, p)
    return (adj_pred, class_logits, h)
